# Initial kernel scaffold; baseline (speedup 1.0000x reference)
import jax
import jax.numpy as jnp
from jax.experimental import pallas as pl


def kernel(encoded_tokens, pos_table):
    raise NotImplementedError("write your pallas kernel here")


# speedup vs baseline: 4.6163x; 4.6163x over previous
import jax
import jax.numpy as jnp
from jax.experimental import pallas as pl

_BLOCK_ROWS = 256


def _bcast_kernel(pos_ref, out_ref):
    out_ref[...] = jnp.broadcast_to(pos_ref[...][None, :, :], out_ref.shape)


def kernel(encoded_tokens, pos_table):
    batch = encoded_tokens.shape[0]
    nrows, dim = pos_table.shape
    grid = (nrows // _BLOCK_ROWS,)
    return pl.pallas_call(
        _bcast_kernel,
        grid=grid,
        in_specs=[pl.BlockSpec((_BLOCK_ROWS, dim), lambda i: (i, 0))],
        out_specs=pl.BlockSpec((batch, _BLOCK_ROWS, dim), lambda i: (0, i, 0)),
        out_shape=jax.ShapeDtypeStruct((batch, nrows, dim), pos_table.dtype),
    )(pos_table)
